# baseline (device time: 179803 ns/iter reference)
import jax
import jax.numpy as jnp
from jax import lax
from jax.experimental import pallas as pl
from jax.experimental.pallas import tpu as pltpu

N_DEV = 8
N_RING = 4
CW = (True, True, False, False)


def kernel(x):
    m, n = x.shape[1], x.shape[2]
    mc = m // N_DEV
    nq = n // N_RING

    def body(x_ref, out_ref, sb, rv, gt, xs, sems, load_sems, out_sems):
        me = lax.axis_index("i")
        right = (me + 1) % N_DEV
        left = (me - 1) % N_DEV

        def col(r):
            return pl.ds(r * nq, nq)

        def tgt(r):
            return right if CW[r] else left

        def add_chunk(r, s):
            return (me - s - 1) % N_DEV if CW[r] else (me + s + 1) % N_DEV

        def ag_chunk(r, t):
            return (me + 1 - t) % N_DEV if CW[r] else (me - 1 + t) % N_DEV

        def load(r, c, slot):
            cp = pltpu.make_async_copy(
                src_ref=x_ref.at[0, pl.ds(c * mc, mc), col(r)],
                dst_ref=xs.at[r, slot],
                sem=load_sems.at[r, slot],
            )
            cp.start()
            return cp

        def store_out(r, c, k):
            sl = (pl.ds(c * mc, mc), col(r))
            cp = pltpu.make_async_copy(
                src_ref=gt.at[sl], dst_ref=out_ref.at[sl],
                sem=out_sems.at[r, k],
            )
            cp.start()
            return cp

        def rs_rdma(r, s):
            return pltpu.make_async_remote_copy(
                src_ref=sb.at[r],
                dst_ref=rv.at[r, s],
                send_sem=sems.at[0, r, s, 0],
                recv_sem=sems.at[0, r, s, 1],
                device_id=(tgt(r),),
                device_id_type=pl.DeviceIdType.MESH,
            )

        def ag_rdma(r, t):
            c = ag_chunk(r, t)
            sl = (pl.ds(c * mc, mc), col(r))
            return pltpu.make_async_remote_copy(
                src_ref=gt.at[sl],
                dst_ref=gt.at[sl],
                send_sem=sems.at[1, r, t, 0],
                recv_sem=sems.at[1, r, t, 1],
                device_id=(tgt(r),),
                device_id_type=pl.DeviceIdType.MESH,
            )

        ORDER = (0, 2, 1, 3)

        l0 = [load(r, me, 0) for r in range(N_RING)]
        l1 = [load(r, add_chunk(r, 0), 1) for r in range(N_RING)]

        barrier_sem = pltpu.get_barrier_semaphore()
        for nbr in (left, right):
            pl.semaphore_signal(
                barrier_sem, inc=1,
                device_id=(nbr,), device_id_type=pl.DeviceIdType.MESH,
            )
        pl.semaphore_wait(barrier_sem, 2)

        cur = [None] * N_RING
        for r in ORDER:
            l0[r].wait()
            sb[r] = xs[r, 0].astype(jnp.bfloat16)
            cur[r] = rs_rdma(r, 0)
            cur[r].start()
        pending = l1

        ag_cur = [None] * N_RING
        stores = []
        for s in range(N_DEV - 1):
            for r in ORDER:
                if s < N_DEV - 2:
                    nl = load(r, add_chunk(r, s + 1), s % 2)
                cur[r].wait()
                pending[r].wait()
                sb[r] = rv[r, s] + xs[r, (s + 1) % 2].astype(jnp.bfloat16)
                if s < N_DEV - 2:
                    cur[r] = rs_rdma(r, s + 1)
                    cur[r].start()
                    pending[r] = nl
                else:
                    gt[pl.ds(ag_chunk(r, 0) * mc, mc), col(r)] = sb[r]
                    ag_cur[r] = ag_rdma(r, 0)
                    ag_cur[r].start()
                    stores.append(store_out(r, ag_chunk(r, 0), 0))

        for t in range(N_DEV - 1):
            for r in ORDER:
                ag_cur[r].wait()
                if t < N_DEV - 2:
                    ag_cur[r] = ag_rdma(r, t + 1)
                    ag_cur[r].start()
                stores.append(store_out(r, ag_chunk(r, t + 1), t + 1))

        for cp in stores:
            cp.wait()

    out_shape = jax.ShapeDtypeStruct((m, n), jnp.bfloat16)
    return pl.pallas_call(
        body,
        out_shape=out_shape,
        in_specs=[pl.BlockSpec(memory_space=pl.ANY)],
        out_specs=pl.BlockSpec(memory_space=pl.ANY),
        scratch_shapes=[
            pltpu.VMEM((N_RING, mc, nq), jnp.bfloat16),
            pltpu.VMEM((N_RING, N_DEV - 1, mc, nq), jnp.bfloat16),
            pltpu.VMEM((m, n), jnp.bfloat16),
            pltpu.VMEM((N_RING, 2, mc, nq), jnp.float32),
            pltpu.SemaphoreType.DMA((2, N_RING, N_DEV - 1, 2)),
            pltpu.SemaphoreType.DMA((N_RING, 2)),
            pltpu.SemaphoreType.DMA((N_RING, N_DEV)),
        ],
        compiler_params=pltpu.CompilerParams(
            collective_id=0, vmem_limit_bytes=100 * 1024 * 1024
        ),
    )(x)
